# baseline (speedup 1.0000x reference)
import functools

import jax
import jax.numpy as jnp
from jax import lax
from jax.experimental import pallas as pl
from jax.experimental.pallas import tpu as pltpu
from jax.experimental.pallas import tpu_sc as plsc

CUTOFF = 20.0
GAP = 0.1
N_CENTERS = 200

N_NODES = 50000
N_EDGES = 800000

ROW = 16
NC, NS, L = 2, 16, 16
NW = NC * NS
EPW = N_EDGES // NW
CHUNK = 1000
N_CHUNKS = EPW // CHUNK


def _sc_dists_body(postab, src, dst, off, d2out,
                   sidx_v, didx_v, srows_v, drows_v,
                   off_v, d2_v, sem_s, sem_d):
    wid = lax.axis_index("s") * NC + lax.axis_index("c")
    tile_base = wid * EPW

    lanes = lax.iota(jnp.int32, L)

    def chunk_body(ci, _):
        base = tile_base + ci * CHUNK
        pltpu.sync_copy(src.at[pl.ds(base, CHUNK)], sidx_v)
        pltpu.sync_copy(dst.at[pl.ds(base, CHUNK)], didx_v)
        cps = pltpu.async_copy(postab.at[sidx_v], srows_v, sem_s)
        cpd = pltpu.async_copy(postab.at[didx_v], drows_v, sem_d)
        pltpu.sync_copy(off.at[pl.ds(3 * base, 3 * CHUNK)], off_v)
        cps.wait()
        cpd.wait()

        def group(off):
            rows = off + lanes
            col0 = jnp.zeros((L,), jnp.int32)
            sx = plsc.load_gather(srows_v, [rows, col0])
            sy = plsc.load_gather(srows_v, [rows, col0 + 1])
            sz = plsc.load_gather(srows_v, [rows, col0 + 2])
            dx = plsc.load_gather(drows_v, [rows, col0])
            dy = plsc.load_gather(drows_v, [rows, col0 + 1])
            dz = plsc.load_gather(drows_v, [rows, col0 + 2])
            o3 = rows * 3
            ox = plsc.load_gather(off_v, [o3])
            oy = plsc.load_gather(off_v, [o3 + 1])
            oz = plsc.load_gather(off_v, [o3 + 2])
            ex = sx - dx - ox
            ey = sy - dy - oy
            ez = sz - dz - oz
            d2_v[pl.ds(off, L)] = ex * ex + ey * ey + ez * ez

        def group_body(g, _):
            group(g * L)
            return 0

        lax.fori_loop(0, CHUNK // L - 1, group_body, 0)
        group(CHUNK - L)

        pltpu.sync_copy(d2_v, d2out.at[pl.ds(base, CHUNK)])
        return 0

    lax.fori_loop(0, N_CHUNKS, chunk_body, 0)


_sc_dists = functools.partial(
    pl.kernel,
    out_type=jax.ShapeDtypeStruct((N_EDGES,), jnp.float32),
    mesh=plsc.VectorSubcoreMesh(core_axis_name="c", subcore_axis_name="s"),
    scratch_types=[
        pltpu.VMEM((CHUNK,), jnp.int32),
        pltpu.VMEM((CHUNK,), jnp.int32),
        pltpu.VMEM((CHUNK, ROW), jnp.float32),
        pltpu.VMEM((CHUNK, ROW), jnp.float32),
        pltpu.VMEM((3 * CHUNK,), jnp.float32),
        pltpu.VMEM((CHUNK,), jnp.float32),
        pltpu.SemaphoreType.DMA,
        pltpu.SemaphoreType.DMA,
    ],
    compiler_params=pltpu.CompilerParams(needs_layout_passes=False,
                                         use_tc_tiling_on_sc=False),
)(_sc_dists_body)


BE = 8192


def _tc_expand_body(d2_ref, centers_ref, out_ref):
    d = jnp.sqrt(d2_ref[...]).reshape(BE, 1)
    t = d - centers_ref[...]
    out_ref[...] = jnp.exp(-(t * t) / GAP)


def _tc_expand(d2, centers):
    return pl.pallas_call(
        _tc_expand_body,
        grid=(pl.cdiv(N_EDGES, BE),),
        in_specs=[
            pl.BlockSpec((BE,), lambda i: (i,)),
            pl.BlockSpec((1, N_CENTERS), lambda i: (0, 0)),
        ],
        out_specs=pl.BlockSpec((BE, N_CENTERS), lambda i: (i, 0)),
        out_shape=jax.ShapeDtypeStruct((N_EDGES, N_CENTERS), jnp.float32),
    )(d2, centers)


@jax.jit
def kernel(position, edge_index, offset):
    postab = jnp.pad(position, ((0, 0), (0, ROW - 3)))
    src = edge_index[0]
    dst = edge_index[1]
    off_flat = offset.reshape(3 * N_EDGES)
    d2 = _sc_dists(postab, src, dst, off_flat)
    centers = jnp.linspace(0.0, CUTOFF, N_CENTERS,
                           dtype=jnp.float32)[None, :]
    return _tc_expand(d2, centers)


# speedup vs baseline: 3.1922x; 3.1922x over previous
import functools

import jax
import jax.numpy as jnp
from jax import lax
from jax.experimental import pallas as pl
from jax.experimental.pallas import tpu as pltpu
from jax.experimental.pallas import tpu_sc as plsc

CUTOFF = 20.0
GAP = 0.1
N_CENTERS = 200

N_NODES = 50000
N_EDGES = 800000

ROW = 16
NC, NS, L = 2, 16, 16
NW = NC * NS
EPW = N_EDGES // NW
CHUNK = 1000
N_CHUNKS = EPW // CHUNK


def _sc_dists_body(postab, src, dst, offx, offy, offz, d2out,
                   sidx_v, didx_v, srows_v, drows_v,
                   ox_v, oy_v, oz_v, d2_v, sem_s, sem_d):
    wid = lax.axis_index("s") * NC + lax.axis_index("c")
    tile_base = wid * EPW

    lanes = lax.iota(jnp.int32, L)

    def chunk_body(ci, _):
        base = tile_base + ci * CHUNK
        pltpu.sync_copy(src.at[pl.ds(base, CHUNK)], sidx_v)
        pltpu.sync_copy(dst.at[pl.ds(base, CHUNK)], didx_v)
        cps = pltpu.async_copy(postab.at[sidx_v], srows_v, sem_s)
        cpd = pltpu.async_copy(postab.at[didx_v], drows_v, sem_d)
        pltpu.sync_copy(offx.at[pl.ds(base, CHUNK)], ox_v)
        pltpu.sync_copy(offy.at[pl.ds(base, CHUNK)], oy_v)
        pltpu.sync_copy(offz.at[pl.ds(base, CHUNK)], oz_v)
        cps.wait()
        cpd.wait()

        def group(off):
            rows = off + lanes
            col0 = jnp.zeros((L,), jnp.int32)
            sx = plsc.load_gather(srows_v, [rows, col0])
            sy = plsc.load_gather(srows_v, [rows, col0 + 1])
            sz = plsc.load_gather(srows_v, [rows, col0 + 2])
            dx = plsc.load_gather(drows_v, [rows, col0])
            dy = plsc.load_gather(drows_v, [rows, col0 + 1])
            dz = plsc.load_gather(drows_v, [rows, col0 + 2])
            ex = sx - dx - ox_v[pl.ds(off, L)]
            ey = sy - dy - oy_v[pl.ds(off, L)]
            ez = sz - dz - oz_v[pl.ds(off, L)]
            d2_v[pl.ds(off, L)] = ex * ex + ey * ey + ez * ez

        def group_body(g, _):
            group(g * L)
            return 0

        lax.fori_loop(0, CHUNK // L - 1, group_body, 0)
        group(CHUNK - L)

        pltpu.sync_copy(d2_v, d2out.at[pl.ds(base, CHUNK)])
        return 0

    lax.fori_loop(0, N_CHUNKS, chunk_body, 0)


_sc_dists = functools.partial(
    pl.kernel,
    out_type=jax.ShapeDtypeStruct((N_EDGES,), jnp.float32),
    mesh=plsc.VectorSubcoreMesh(core_axis_name="c", subcore_axis_name="s"),
    scratch_types=[
        pltpu.VMEM((CHUNK,), jnp.int32),
        pltpu.VMEM((CHUNK,), jnp.int32),
        pltpu.VMEM((CHUNK, ROW), jnp.float32),
        pltpu.VMEM((CHUNK, ROW), jnp.float32),
        pltpu.VMEM((CHUNK,), jnp.float32),
        pltpu.VMEM((CHUNK,), jnp.float32),
        pltpu.VMEM((CHUNK,), jnp.float32),
        pltpu.VMEM((CHUNK,), jnp.float32),
        pltpu.SemaphoreType.DMA,
        pltpu.SemaphoreType.DMA,
    ],
    compiler_params=pltpu.CompilerParams(needs_layout_passes=False,
                                         use_tc_tiling_on_sc=False),
)(_sc_dists_body)


BE = 8192


def _tc_expand_body(d2_ref, centers_ref, out_ref):
    d = jnp.sqrt(d2_ref[...]).reshape(BE, 1)
    t = d - centers_ref[...]
    out_ref[...] = jnp.exp(-(t * t) / GAP)


def _tc_expand(d2, centers):
    return pl.pallas_call(
        _tc_expand_body,
        grid=(pl.cdiv(N_EDGES, BE),),
        in_specs=[
            pl.BlockSpec((BE,), lambda i: (i,)),
            pl.BlockSpec((1, N_CENTERS), lambda i: (0, 0)),
        ],
        out_specs=pl.BlockSpec((BE, N_CENTERS), lambda i: (i, 0)),
        out_shape=jax.ShapeDtypeStruct((N_EDGES, N_CENTERS), jnp.float32),
    )(d2, centers)


@jax.jit
def kernel(position, edge_index, offset):
    postab = jnp.pad(position, ((0, 0), (0, ROW - 3)))
    src = edge_index[0]
    dst = edge_index[1]
    d2 = _sc_dists(postab, src, dst,
                   offset[:, 0], offset[:, 1], offset[:, 2])
    centers = jnp.linspace(0.0, CUTOFF, N_CENTERS,
                           dtype=jnp.float32)[None, :]
    return _tc_expand(d2, centers)


# speedup vs baseline: 7.5362x; 2.3608x over previous
import functools

import jax
import jax.numpy as jnp
from jax import lax
from jax.experimental import pallas as pl
from jax.experimental.pallas import tpu as pltpu
from jax.experimental.pallas import tpu_sc as plsc

CUTOFF = 20.0
GAP = 0.1
N_CENTERS = 200

N_NODES = 50000
N_EDGES = 800000

ROW = 16
NC, NS, L = 2, 16, 16
NW = NC * NS
EPW = N_EDGES // NW
CHUNK = 1000
N_CHUNKS = EPW // CHUNK


def _sc_dists_body(postab, src, dst, offx, offy, offz, d2out,
                   sidx_v, didx_v, srows_v, drows_v,
                   ox_v, oy_v, oz_v, d2_v, sem_s, sem_d):
    wid = lax.axis_index("s") * NC + lax.axis_index("c")
    tile_base = wid * EPW

    lanes = lax.iota(jnp.int32, L)

    def chunk_body(ci, _):
        base = tile_base + ci * CHUNK
        pltpu.sync_copy(src.at[pl.ds(base, CHUNK)], sidx_v)
        pltpu.sync_copy(dst.at[pl.ds(base, CHUNK)], didx_v)
        cps = pltpu.async_copy(postab.at[sidx_v], srows_v, sem_s)
        cpd = pltpu.async_copy(postab.at[didx_v], drows_v, sem_d)
        pltpu.sync_copy(offx.at[pl.ds(base, CHUNK)], ox_v)
        pltpu.sync_copy(offy.at[pl.ds(base, CHUNK)], oy_v)
        pltpu.sync_copy(offz.at[pl.ds(base, CHUNK)], oz_v)
        cps.wait()
        cpd.wait()

        def group(off):
            rows = off + lanes
            col0 = jnp.zeros((L,), jnp.int32)
            sx = plsc.load_gather(srows_v, [rows, col0])
            sy = plsc.load_gather(srows_v, [rows, col0 + 1])
            sz = plsc.load_gather(srows_v, [rows, col0 + 2])
            dx = plsc.load_gather(drows_v, [rows, col0])
            dy = plsc.load_gather(drows_v, [rows, col0 + 1])
            dz = plsc.load_gather(drows_v, [rows, col0 + 2])
            ex = sx - dx - ox_v[pl.ds(off, L)]
            ey = sy - dy - oy_v[pl.ds(off, L)]
            ez = sz - dz - oz_v[pl.ds(off, L)]
            d2_v[pl.ds(off, L)] = ex * ex + ey * ey + ez * ez

        def group_body(g, _):
            group(g * L)
            return 0

        lax.fori_loop(0, CHUNK // L - 1, group_body, 0)
        group(CHUNK - L)

        pltpu.sync_copy(d2_v, d2out.at[pl.ds(base, CHUNK)])
        return 0

    lax.fori_loop(0, N_CHUNKS, chunk_body, 0)


_sc_dists = functools.partial(
    pl.kernel,
    out_type=jax.ShapeDtypeStruct((N_EDGES,), jnp.float32),
    mesh=plsc.VectorSubcoreMesh(core_axis_name="c", subcore_axis_name="s"),
    scratch_types=[
        pltpu.VMEM((CHUNK,), jnp.int32),
        pltpu.VMEM((CHUNK,), jnp.int32),
        pltpu.VMEM((CHUNK, ROW), jnp.float32),
        pltpu.VMEM((CHUNK, ROW), jnp.float32),
        pltpu.VMEM((CHUNK,), jnp.float32),
        pltpu.VMEM((CHUNK,), jnp.float32),
        pltpu.VMEM((CHUNK,), jnp.float32),
        pltpu.VMEM((CHUNK,), jnp.float32),
        pltpu.SemaphoreType.DMA,
        pltpu.SemaphoreType.DMA,
    ],
    compiler_params=pltpu.CompilerParams(needs_layout_passes=False,
                                         use_tc_tiling_on_sc=False),
)(_sc_dists_body)


BE = 8192


def _tc_expand_body(d2_ref, centers_ref, out_ref):
    d = jnp.sqrt(d2_ref[...])
    t = centers_ref[...] - d[None, :]
    out_ref[...] = jnp.exp(-(t * t) / GAP)


def _tc_expand(d2, centers):
    return pl.pallas_call(
        _tc_expand_body,
        grid=(pl.cdiv(N_EDGES, BE),),
        in_specs=[
            pl.BlockSpec((BE,), lambda i: (i,)),
            pl.BlockSpec((N_CENTERS, 1), lambda i: (0, 0)),
        ],
        out_specs=pl.BlockSpec((N_CENTERS, BE), lambda i: (0, i)),
        out_shape=jax.ShapeDtypeStruct((N_CENTERS, N_EDGES), jnp.float32),
    )(d2, centers)


@jax.jit
def kernel(position, edge_index, offset):
    postab = jnp.pad(position, ((0, 0), (0, ROW - 3)))
    src = edge_index[0]
    dst = edge_index[1]
    d2 = _sc_dists(postab, src, dst,
                   offset[:, 0], offset[:, 1], offset[:, 2])
    centers = jnp.linspace(0.0, CUTOFF, N_CENTERS,
                           dtype=jnp.float32)[:, None]
    return _tc_expand(d2, centers).T


# speedup vs baseline: 7.7828x; 1.0327x over previous
import functools

import jax
import jax.numpy as jnp
from jax import lax
from jax.experimental import pallas as pl
from jax.experimental.pallas import tpu as pltpu
from jax.experimental.pallas import tpu_sc as plsc

CUTOFF = 20.0
GAP = 0.1
N_CENTERS = 200

N_NODES = 50000
N_EDGES = 800000

ROW = 16
NC, NS, L = 2, 16, 16
NW = NC * NS
EPW = N_EDGES // NW
CHUNK = 1000
N_CHUNKS = EPW // CHUNK


def _sc_dists_body(postab, src, dst, offx, offy, offz, d2out,
                   sidx_v, didx_v, srows_v, drows_v,
                   ox_v, oy_v, oz_v, d2_v, sem_s, sem_d):
    wid = lax.axis_index("s") * NC + lax.axis_index("c")
    tile_base = wid * EPW

    lanes = lax.iota(jnp.int32, L)

    def chunk_body(ci, _):
        base = tile_base + ci * CHUNK
        pltpu.sync_copy(src.at[pl.ds(base, CHUNK)], sidx_v)
        pltpu.sync_copy(dst.at[pl.ds(base, CHUNK)], didx_v)
        cps = pltpu.async_copy(postab.at[sidx_v], srows_v, sem_s)
        cpd = pltpu.async_copy(postab.at[didx_v], drows_v, sem_d)
        pltpu.sync_copy(offx.at[pl.ds(base, CHUNK)], ox_v)
        pltpu.sync_copy(offy.at[pl.ds(base, CHUNK)], oy_v)
        pltpu.sync_copy(offz.at[pl.ds(base, CHUNK)], oz_v)
        cps.wait()
        cpd.wait()

        def group(off):
            rows = off + lanes
            col0 = jnp.zeros((L,), jnp.int32)
            sx = plsc.load_gather(srows_v, [rows, col0])
            sy = plsc.load_gather(srows_v, [rows, col0 + 1])
            sz = plsc.load_gather(srows_v, [rows, col0 + 2])
            dx = plsc.load_gather(drows_v, [rows, col0])
            dy = plsc.load_gather(drows_v, [rows, col0 + 1])
            dz = plsc.load_gather(drows_v, [rows, col0 + 2])
            ex = sx - dx - ox_v[pl.ds(off, L)]
            ey = sy - dy - oy_v[pl.ds(off, L)]
            ez = sz - dz - oz_v[pl.ds(off, L)]
            d2_v[pl.ds(off, L)] = ex * ex + ey * ey + ez * ez

        def group_body(g, _):
            group(g * L)
            return 0

        lax.fori_loop(0, CHUNK // L - 1, group_body, 0)
        group(CHUNK - L)

        pltpu.sync_copy(d2_v, d2out.at[pl.ds(base, CHUNK)])
        return 0

    lax.fori_loop(0, N_CHUNKS, chunk_body, 0)


_sc_dists = functools.partial(
    pl.kernel,
    out_type=jax.ShapeDtypeStruct((N_EDGES,), jnp.float32),
    mesh=plsc.VectorSubcoreMesh(core_axis_name="c", subcore_axis_name="s"),
    scratch_types=[
        pltpu.VMEM((CHUNK,), jnp.int32),
        pltpu.VMEM((CHUNK,), jnp.int32),
        pltpu.VMEM((CHUNK, ROW), jnp.float32),
        pltpu.VMEM((CHUNK, ROW), jnp.float32),
        pltpu.VMEM((CHUNK,), jnp.float32),
        pltpu.VMEM((CHUNK,), jnp.float32),
        pltpu.VMEM((CHUNK,), jnp.float32),
        pltpu.VMEM((CHUNK,), jnp.float32),
        pltpu.SemaphoreType.DMA,
        pltpu.SemaphoreType.DMA,
    ],
    compiler_params=pltpu.CompilerParams(needs_layout_passes=False,
                                         use_tc_tiling_on_sc=False),
)(_sc_dists_body)


BE = 16384


def _tc_expand_body(d2_ref, centers_ref, out_ref):
    d = jnp.sqrt(d2_ref[...])
    t = centers_ref[...] - d[None, :]
    out_ref[...] = jnp.exp(-(t * t) / GAP)


def _tc_expand(d2, centers):
    return pl.pallas_call(
        _tc_expand_body,
        grid=(pl.cdiv(N_EDGES, BE),),
        in_specs=[
            pl.BlockSpec((BE,), lambda i: (i,)),
            pl.BlockSpec((N_CENTERS, 1), lambda i: (0, 0)),
        ],
        out_specs=pl.BlockSpec((N_CENTERS, BE), lambda i: (0, i)),
        out_shape=jax.ShapeDtypeStruct((N_CENTERS, N_EDGES), jnp.float32),
    )(d2, centers)


@jax.jit
def kernel(position, edge_index, offset):
    postab = jnp.pad(position, ((0, 0), (0, ROW - 3)))
    src = edge_index[0]
    dst = edge_index[1]
    d2 = _sc_dists(postab, src, dst,
                   offset[:, 0], offset[:, 1], offset[:, 2])
    centers = jnp.linspace(0.0, CUTOFF, N_CENTERS,
                           dtype=jnp.float32)[:, None]
    return _tc_expand(d2, centers).T
